# baseline (speedup 1.0000x reference)
import jax
import jax.numpy as jnp
from jax import lax
from jax.experimental import pallas as pl
from jax.experimental.pallas import tpu as pltpu, tpu_sc as plsc

N_ROWS = 1000001
K = 64
T = 50
B = 16384

NC = 2
NS = 16
NW = NC * NS
CHUNK = B // NW
G = 16
NG = CHUNK // G
KV = K // 16


def _body(users_hbm, items_hbm, seq_hbm, vui_hbm, viu_hbm, vil_hbm,
          vli_hbm, out_hbm,
          users_s, items_s, seq_s, idx_v, seqv_v, stage_v, u_v, iu_v, il_v,
          out_v, sem_hist, sem_rows):
    wid = lax.axis_index("s") * NC + lax.axis_index("c")
    base = wid * CHUNK

    pltpu.sync_copy(users_hbm.at[pl.ds(base, CHUNK)], idx_v)

    @pl.loop(0, CHUNK // 16)
    def _stage_u(q):
        v = idx_v[pl.ds(pl.multiple_of(q * 16, 16), 16)]
        for j in range(16):
            users_s[q * 16 + j] = jax.lax.index_in_dim(v, j, 0, False)

    pltpu.sync_copy(items_hbm.at[pl.ds(base, CHUNK)], idx_v)

    @pl.loop(0, CHUNK // 16)
    def _stage_i(q):
        v = idx_v[pl.ds(pl.multiple_of(q * 16, 16), 16)]
        for j in range(16):
            items_s[q * 16 + j] = jax.lax.index_in_dim(v, j, 0, False)

    @pl.loop(0, NG)
    def _grp(g):
        off = pl.multiple_of(g * G, G)

        pltpu.sync_copy(seq_hbm.at[pl.ds((base + off) * T, G * T)], seqv_v)

        @pl.loop(0, (G * T) // 16)
        def _stage_s(q):
            v = seqv_v[pl.ds(pl.multiple_of(q * 16, 16), 16)]
            for j in range(16):
                seq_s[q * 16 + j] = jax.lax.index_in_dim(v, j, 0, False)

        @pl.loop(0, G)
        def _fire(r):
            @pl.loop(0, T)
            def _hist(t):
                s = seq_s[r * T + t]
                pltpu.async_copy(vli_hbm.at[pl.ds(s * K, K)],
                                 stage_v.at[pl.ds((r * T + t) * K, K)],
                                 sem_hist)

            u = users_s[off + r]
            i = items_s[off + r]
            pltpu.async_copy(vui_hbm.at[pl.ds(u * K, K)],
                             u_v.at[pl.ds(r * K, K)], sem_rows)
            pltpu.async_copy(viu_hbm.at[pl.ds(i * K, K)],
                             iu_v.at[pl.ds(r * K, K)], sem_rows)
            pltpu.async_copy(vil_hbm.at[pl.ds(i * K, K)],
                             il_v.at[pl.ds(r * K, K)], sem_rows)

        pltpu.make_async_copy(vli_hbm.at[pl.ds(0, G * T * K)], stage_v,
                              sem_hist).wait()
        pltpu.make_async_copy(vui_hbm.at[pl.ds(0, G * K)], u_v,
                              sem_rows).wait()
        pltpu.make_async_copy(vui_hbm.at[pl.ds(0, G * K)], iu_v,
                              sem_rows).wait()
        pltpu.make_async_copy(vui_hbm.at[pl.ds(0, G * K)], il_v,
                              sem_rows).wait()

        lanes = lax.iota(jnp.int32, 16)
        zero16 = jnp.zeros((16,), jnp.float32)

        @pl.loop(0, G, init_carry=zero16)
        def score_vec(j, sc):
            @pl.loop(0, T, init_carry=jnp.int32(0))
            def cnt(t, c):
                return c + jnp.where(seq_s[j * T + t] != 0, 1, 0)

            inv = 1.0 / jnp.maximum(zero16 + cnt.astype(jnp.float32), 1.0)

            il = [il_v[pl.ds(j * K + k * 16, 16)] for k in range(KV)]
            s_ui = zero16
            for k in range(KV):
                s_ui = s_ui + (u_v[pl.ds(j * K + k * 16, 16)]
                               * iu_v[pl.ds(j * K + k * 16, 16)])

            @pl.loop(0, T, init_carry=tuple(zero16 for _ in range(KV)))
            def s_il(t, carry):
                rb = (j * T + t) * K
                return tuple(
                    carry[k] + il[k] * stage_v[pl.ds(rb + k * 16, 16)]
                    for k in range(KV))

            s_il_tot = s_il[0]
            for k in range(1, KV):
                s_il_tot = s_il_tot + s_il[k]

            onehot = jnp.where(lanes == j, 1.0, 0.0).astype(jnp.float32)
            return (sc + jnp.sum(s_ui) * onehot
                    + (jnp.sum(s_il_tot) * onehot) * inv)

        out_v[pl.ds(off, G)] = score_vec

    pltpu.sync_copy(out_v, out_hbm.at[pl.ds(base, CHUNK)])


@jax.jit
def kernel(users, items, seq_padded, VUI, VIU, VIL, VLI):
    seq_flat = jnp.asarray(seq_padded, jnp.int32).reshape(B * T)
    users = jnp.asarray(users, jnp.int32)
    items = jnp.asarray(items, jnp.int32)

    call = pl.kernel(
        _body,
        out_type=jax.ShapeDtypeStruct((B,), jnp.float32),
        mesh=plsc.VectorSubcoreMesh(core_axis_name="c", subcore_axis_name="s"),
        compiler_params=pltpu.CompilerParams(use_tc_tiling_on_sc=True,
                                             needs_layout_passes=False),
        scratch_types=[
            pltpu.SMEM((CHUNK,), jnp.int32),
            pltpu.SMEM((CHUNK,), jnp.int32),
            pltpu.SMEM((G * T,), jnp.int32),
            pltpu.VMEM((CHUNK,), jnp.int32),
            pltpu.VMEM((G * T,), jnp.int32),
            pltpu.VMEM((G * T * K,), jnp.float32),
            pltpu.VMEM((G * K,), jnp.float32),
            pltpu.VMEM((G * K,), jnp.float32),
            pltpu.VMEM((G * K,), jnp.float32),
            pltpu.VMEM((CHUNK,), jnp.float32),
            pltpu.SemaphoreType.DMA,
            pltpu.SemaphoreType.DMA,
        ],
    )
    return call(users, items, seq_flat,
                VUI.reshape(N_ROWS * K), VIU.reshape(N_ROWS * K),
                VIL.reshape(N_ROWS * K), VLI.reshape(N_ROWS * K))


# speedup vs baseline: 1.4520x; 1.4520x over previous
import functools
import jax
import jax.numpy as jnp
from jax import lax
from jax.experimental import pallas as pl
from jax.experimental.pallas import tpu as pltpu, tpu_sc as plsc

N_ROWS = 1000001
K = 64
T = 50
B = 16384

NC = 2
NS = 16
NW = NC * NS
CHUNK = B // NW
SUB = 128
NSUB = CHUNK // SUB
KV = K // 16


def _ctx_body(seqT_hbm, vli_hbm, ctx_hbm, cnt_hbm,
              seq_v, acc_v, out_v, cnt_v, sem_acc):
    wid = lax.axis_index("s") * NC + lax.axis_index("c")
    base = wid * CHUNK

    pltpu.sync_copy(seqT_hbm.at[:, pl.ds(base, CHUNK)], seq_v)

    @pl.loop(0, NSUB)
    def _sub(sub):
        off = pl.multiple_of(sub * SUB, SUB)

        @pl.loop(0, SUB)
        def _zero(r):
            for k in range(KV):
                acc_v[r, pl.ds(k * 16, 16)] = jnp.zeros((16,), jnp.float32)

        @pl.loop(0, T)
        def _fire(t):
            pltpu.async_copy(vli_hbm.at[seq_v.at[t, pl.ds(off, SUB)]],
                             acc_v, sem_acc, add=True)

        @pl.loop(0, SUB // 16)
        def _cnt(g):
            @pl.loop(0, T, init_carry=jnp.zeros((16,), jnp.float32))
            def cnt(t, c):
                s = seq_v[t, pl.ds(pl.multiple_of(off + g * 16, 16), 16)]
                return c + jnp.where(s != 0, 1.0, 0.0).astype(jnp.float32)

            cnt_v[pl.ds(pl.multiple_of(off + g * 16, 16), 16)] = cnt

        @pl.loop(0, T)
        def _drain(t):
            pltpu.make_async_copy(vli_hbm.at[pl.ds(0, SUB)],
                                  acc_v, sem_acc).wait()

        @pl.loop(0, SUB)
        def _flat(r):
            for k in range(KV):
                out_v[pl.ds((off + r) * K + k * 16, 16)] = \
                    acc_v[r, pl.ds(k * 16, 16)]

    pltpu.sync_copy(out_v, ctx_hbm.at[pl.ds(base * K, CHUNK * K)])
    pltpu.sync_copy(cnt_v, cnt_hbm.at[pl.ds(base, CHUNK)])


def _score_body(users_hbm, items_hbm, vui_hbm, viu_hbm, vil_hbm,
                ctx_hbm, cnt_hbm, out_hbm,
                users_s, items_s, idx_v, idx2_v, u_v, iu_v, il_v, ctx_v,
                cnt_v, out_v, sem_rows):
    wid = lax.axis_index("s") * NC + lax.axis_index("c")
    base = wid * CHUNK

    pltpu.sync_copy(users_hbm.at[pl.ds(base, CHUNK)], idx_v)
    pltpu.sync_copy(items_hbm.at[pl.ds(base, CHUNK)], idx2_v)
    pltpu.sync_copy(cnt_hbm.at[pl.ds(base, CHUNK)], cnt_v)

    @pl.loop(0, CHUNK // 16)
    def _stage(g):
        uv = idx_v[pl.ds(pl.multiple_of(g * 16, 16), 16)]
        iv = idx2_v[pl.ds(pl.multiple_of(g * 16, 16), 16)]
        for j in range(16):
            users_s[g * 16 + j] = jax.lax.index_in_dim(uv, j, 0, False)
            items_s[g * 16 + j] = jax.lax.index_in_dim(iv, j, 0, False)

    @pl.loop(0, NSUB)
    def _sub(sub):
        off = pl.multiple_of(sub * SUB, SUB)

        pltpu.async_copy(ctx_hbm.at[pl.ds((base + off) * K, SUB * K)],
                         ctx_v, sem_rows)

        @pl.loop(0, SUB)
        def _fetch(r):
            u = users_s[off + r]
            i = items_s[off + r]
            pltpu.async_copy(vui_hbm.at[pl.ds(u, 1)], u_v.at[pl.ds(r, 1)],
                             sem_rows)
            pltpu.async_copy(viu_hbm.at[pl.ds(i, 1)], iu_v.at[pl.ds(r, 1)],
                             sem_rows)
            pltpu.async_copy(vil_hbm.at[pl.ds(i, 1)], il_v.at[pl.ds(r, 1)],
                             sem_rows)

        pltpu.make_async_copy(ctx_hbm.at[pl.ds(0, SUB * K)], ctx_v,
                              sem_rows).wait()
        pltpu.make_async_copy(vui_hbm.at[pl.ds(0, SUB)], u_v, sem_rows).wait()
        pltpu.make_async_copy(vui_hbm.at[pl.ds(0, SUB)], iu_v, sem_rows).wait()
        pltpu.make_async_copy(vui_hbm.at[pl.ds(0, SUB)], il_v, sem_rows).wait()

        @pl.loop(0, SUB // 16)
        def _grp(g):
            lanes = lax.iota(jnp.int32, 16)
            cnt = cnt_v[pl.ds(pl.multiple_of(off + g * 16, 16), 16)]
            inv = 1.0 / jnp.maximum(cnt, 1.0)

            zero16 = jnp.zeros((16,), jnp.float32)

            @pl.loop(0, 16, init_carry=(zero16, zero16))
            def dots(j, carry):
                s_ui_acc, s_il_acc = carry
                r = g * 16 + j
                s_ui = zero16
                s_il = zero16
                for k in range(KV):
                    ks = pl.ds(k * 16, 16)
                    s_ui = s_ui + u_v[r, ks] * iu_v[r, ks]
                    s_il = s_il + il_v[r, ks] * ctx_v[pl.ds(r * K + k * 16, 16)]
                onehot = jnp.where(lanes == j, 1.0, 0.0).astype(jnp.float32)
                return (s_ui_acc + jnp.sum(s_ui) * onehot,
                        s_il_acc + jnp.sum(s_il) * onehot)

            score = dots[0] + dots[1] * inv
            out_v[pl.ds(pl.multiple_of(off + g * 16, 16), 16)] = score

    pltpu.sync_copy(out_v, out_hbm.at[pl.ds(base, CHUNK)])


@jax.jit
def kernel(users, items, seq_padded, VUI, VIU, VIL, VLI):
    seq_t = jnp.asarray(seq_padded, jnp.int32).T
    users = jnp.asarray(users, jnp.int32)
    items = jnp.asarray(items, jnp.int32)

    mesh = plsc.VectorSubcoreMesh(core_axis_name="c", subcore_axis_name="s")

    ctx_call = pl.kernel(
        _ctx_body,
        out_type=[jax.ShapeDtypeStruct((B * K,), jnp.float32),
                  jax.ShapeDtypeStruct((B,), jnp.float32)],
        mesh=mesh,
        compiler_params=pltpu.CompilerParams(use_tc_tiling_on_sc=False,
                                             needs_layout_passes=False),
        scratch_types=[
            pltpu.VMEM((T, CHUNK), jnp.int32),
            pltpu.VMEM((SUB, K), jnp.float32),
            pltpu.VMEM((CHUNK * K,), jnp.float32),
            pltpu.VMEM((CHUNK,), jnp.float32),
            pltpu.SemaphoreType.DMA,
        ],
    )
    ctx_sum, counts = ctx_call(seq_t, VLI)

    score_call = pl.kernel(
        _score_body,
        out_type=jax.ShapeDtypeStruct((B,), jnp.float32),
        mesh=mesh,
        compiler_params=pltpu.CompilerParams(use_tc_tiling_on_sc=True,
                                             needs_layout_passes=False),
        scratch_types=[
            pltpu.SMEM((CHUNK,), jnp.int32),
            pltpu.SMEM((CHUNK,), jnp.int32),
            pltpu.VMEM((CHUNK,), jnp.int32),
            pltpu.VMEM((CHUNK,), jnp.int32),
            pltpu.VMEM((SUB, K), jnp.float32),
            pltpu.VMEM((SUB, K), jnp.float32),
            pltpu.VMEM((SUB, K), jnp.float32),
            pltpu.VMEM((SUB * K,), jnp.float32),
            pltpu.VMEM((CHUNK,), jnp.float32),
            pltpu.VMEM((CHUNK,), jnp.float32),
            pltpu.SemaphoreType.DMA,
        ],
    )
    return score_call(users, items, VUI, VIU, VIL, ctx_sum, counts)
